# Initial kernel scaffold; baseline (speedup 1.0000x reference)
import jax
import jax.numpy as jnp
from jax.experimental import pallas as pl


def kernel(x, fc1_w, fc1_b, fc1_g, fc1_beta, mr_w, mr_b, mr_g, mr_beta, fc2_w, fc2_b, fc2_g, fc2_beta):
    raise NotImplementedError("write your pallas kernel here")


# speedup vs baseline: 10262.0132x; 10262.0132x over previous
import math

import jax
import jax.numpy as jnp
import numpy as np
from jax.experimental import pallas as pl

_B, _C, _H, _W = 8, 96, 56, 56
_K = 9
_HR, _WR = 7, 7
_N = _H * _W
_NR = _HR * _WR
_GROUPS = 4
_BIG = 3.0e38


def _pos_embed_np(c, h, w):
    d = c // 2
    pe = np.zeros((c, h, w), dtype=np.float32)
    div = np.exp(np.arange(0.0, d, 2) * -(math.log(10000.0) / d))
    pos_w = np.arange(0.0, w)[:, None]
    pos_h = np.arange(0.0, h)[:, None]
    pe[0:d:2, :, :] = np.sin(pos_w * div).T[:, None, :]
    pe[1:d:2, :, :] = np.cos(pos_w * div).T[:, None, :]
    pe[d::2, :, :] = np.sin(pos_h * div).T[:, :, None]
    pe[d + 1::2, :, :] = np.cos(pos_h * div).T[:, :, None]
    return pe


def _constants():
    pos = _pos_embed_np(_C, _H, _W)
    pos_red = pos.reshape(_C, _HR, _H // _HR, _WR, _W // _WR).mean(axis=(2, 4))
    rel = 2.0 * (pos.reshape(_C, -1).T @ pos_red.reshape(_C, -1)) / _C
    relT = np.ascontiguousarray(rel.T).astype(np.float32)
    hh = np.arange(_H)[:, None]
    ww = np.arange(_W)[None, :]
    blk = (hh // (_H // _HR)) * _WR + (ww // (_W // _WR))
    pool = (blk.reshape(_N, 1) == np.arange(_NR)[None, :]).astype(np.float32) / 64.0
    return relT, pool


_RELT_NP, _POOL_NP = _constants()


def _body(x_ref, w1_ref, b1_ref, mrw_ref, mrb_ref, mrg_ref, mrbeta_ref,
          w2_ref, b2_ref, relT_ref, pool_ref, out_ref):
    x = x_ref[0]
    w1 = w1_ref[...]
    h = jnp.dot(w1, x, preferred_element_type=jnp.float32) + b1_ref[...]

    y = jnp.dot(h, pool_ref[...], preferred_element_type=jnp.float32)

    nx = jnp.sqrt(jnp.sum(h * h, axis=0, keepdims=True))
    ny = jnp.sqrt(jnp.sum(y * y, axis=0, keepdims=True))
    ipT = jax.lax.dot_general(y, h, (((0,), (0,)), ((), ())),
                              preferred_element_type=jnp.float32)
    inv_x = 1.0 / (nx + 1e-12)
    inv_y = 1.0 / (ny + 1e-12)
    innerT = ipT * inv_x * inv_y.reshape(_NR, 1)
    sx = (nx * inv_x) ** 2
    sy = (ny * inv_y) ** 2
    distT = 2.0 * innerT - sx - sy.reshape(_NR, 1) + relT_ref[...]

    iota0 = jax.lax.broadcasted_iota(jnp.int32, (_NR, _N), 0)
    acc = jnp.full((_C, _N), -_BIG, dtype=jnp.float32)
    d = distT
    for _ in range(_K):
        cur = jnp.max(d, axis=0, keepdims=True)
        first = jnp.min(jnp.where(d >= cur, iota0, _NR), axis=0,
                        keepdims=True)
        onehot = iota0 == first
        ysel = jnp.dot(y, onehot.astype(jnp.float32),
                       preferred_element_type=jnp.float32)
        acc = jnp.maximum(acc, ysel)
        d = jnp.where(onehot, -_BIG, d)

    cat = jnp.concatenate([h, acc - h], axis=0)
    g = jnp.dot(mrw_ref[...], cat,
                preferred_element_type=jnp.float32) + mrb_ref[...]

    rows = (2 * _C) // _GROUPS
    parts = []
    for gi in range(_GROUPS):
        sub = g[gi * rows:(gi + 1) * rows, :]
        m = jnp.mean(sub)
        dsub = sub - m
        v = jnp.mean(dsub * dsub)
        parts.append(dsub * jax.lax.rsqrt(v + 1e-5))
    gn = jnp.concatenate(parts, axis=0) * mrg_ref[...] + mrbeta_ref[...]
    act = jax.nn.gelu(gn)

    out = jnp.dot(w2_ref[...], act,
                  preferred_element_type=jnp.float32) + b2_ref[...] + x
    out_ref[0] = out


def kernel(x, fc1_w, fc1_b, fc1_g, fc1_beta, mr_w, mr_b, mr_g, mr_beta,
           fc2_w, fc2_b, fc2_g, fc2_beta):
    x3 = x.reshape(_B, _C, _N)
    w1 = fc1_g[:, None] * fc1_w
    b1 = (fc1_g * fc1_b + fc1_beta)[:, None]
    w2 = fc2_g[:, None] * fc2_w
    b2 = (fc2_g * fc2_b + fc2_beta)[:, None]
    relT = jnp.asarray(_RELT_NP)
    pool = jnp.asarray(_POOL_NP)

    full = lambda shape: pl.BlockSpec(shape, lambda b: (0,) * len(shape))
    out = pl.pallas_call(
        _body,
        grid=(_B,),
        in_specs=[
            pl.BlockSpec((1, _C, _N), lambda b: (b, 0, 0)),
            full((_C, _C)), full((_C, 1)),
            full((2 * _C, 2 * _C)), full((2 * _C, 1)),
            full((2 * _C, 1)), full((2 * _C, 1)),
            full((_C, 2 * _C)), full((_C, 1)),
            full((_NR, _N)), full((_N, _NR)),
        ],
        out_specs=pl.BlockSpec((1, _C, _N), lambda b: (b, 0, 0)),
        out_shape=jax.ShapeDtypeStruct((_B, _C, _N), jnp.float32),
    )(x3, w1, b1, mr_w, mr_b[:, None], mr_g[:, None], mr_beta[:, None],
      w2, b2, relT, pool)
    return out.reshape(_B, _C, _H, _W)
